# Initial kernel scaffold; baseline (speedup 1.0000x reference)
import jax
import jax.numpy as jnp
from jax.experimental import pallas as pl


def kernel(x, unpool_rows, unpool_cols, unpool_vals):
    raise NotImplementedError("write your pallas kernel here")


# speedup vs baseline: 28.3324x; 28.3324x over previous
import jax
import jax.numpy as jnp
from jax import lax
from jax.experimental import pallas as pl
from jax.experimental.pallas import tpu as pltpu
from jax.experimental.pallas import tpu_sc as plsc

N = 65536
NNZ = 4194304
D = 64
L = 16
NC = 2
NS = 16
NCHUNKS = D // L
NPASS = NCHUNKS // NC

CHUNK = 1024
IDXROW = 128
NROWS = CHUNK // IDXROW
TILE_NNZ = NNZ // NS
NITER = TILE_NNZ // CHUNK
ROWS_PER_TILE = N // NS

GSETS = 2
ISETS = 4


def _body(xc, cols2, rows2, vals, out,
          cb0, cb1, cb2, cb3, rb0, rb1, rb2, rb3, vb0, vb1, vb2, vb3,
          gb0, gb1, acc,
          sl0, sl1, sl2, sl3, sg0, sg1, ss0, ss1):
    c = lax.axis_index("c")
    s = lax.axis_index("s")

    cbuf = [cb0, cb1, cb2, cb3]
    rbuf = [rb0, rb1, rb2, rb3]
    vbuf = [vb0, vb1, vb2, vb3]
    gbuf = [gb0, gb1]
    sem_ld = [sl0, sl1, sl2, sl3]
    sem_g = [sg0, sg1]
    sem_sc = [ss0, ss1]

    def loads_fire(k, i4):
        base = pl.multiple_of(s * TILE_NNZ + k * CHUNK, CHUNK)
        brow = pl.multiple_of(base // IDXROW, 8)
        pltpu.async_copy(cols2.at[pl.ds(brow, NROWS)], cbuf[i4], sem_ld[i4])
        pltpu.async_copy(rows2.at[pl.ds(brow, NROWS)], rbuf[i4], sem_ld[i4])
        pltpu.async_copy(vals.at[pl.ds(base, CHUNK)], vbuf[i4], sem_ld[i4])

    def loads_wait(i4):
        pltpu.make_async_copy(cols2.at[pl.ds(0, NROWS)], cbuf[i4], sem_ld[i4]).wait()
        pltpu.make_async_copy(rows2.at[pl.ds(0, NROWS)], rbuf[i4], sem_ld[i4]).wait()
        pltpu.make_async_copy(vals.at[pl.ds(0, CHUNK)], vbuf[i4], sem_ld[i4]).wait()

    def gathers_fire(j, g, i4):
        for r in range(NROWS):
            pltpu.async_copy(xc.at[j].at[cbuf[i4].at[r]],
                             gbuf[g].at[pl.ds(r * IDXROW, IDXROW)], sem_g[g])

    def gathers_wait(j, g, i4):
        for r in range(NROWS):
            pltpu.make_async_copy(xc.at[j].at[cbuf[i4].at[r]],
                                  gbuf[g].at[pl.ds(r * IDXROW, IDXROW)],
                                  sem_g[g]).wait()

    def scatters_fire(g, i4):
        for r in range(NROWS):
            pltpu.async_copy(gbuf[g].at[pl.ds(r * IDXROW, IDXROW)],
                             acc.at[rbuf[i4].at[r]], sem_sc[g], add=True)

    def scatters_wait(g, i4):
        for r in range(NROWS):
            pltpu.make_async_copy(gbuf[g].at[pl.ds(r * IDXROW, IDXROW)],
                                  acc.at[rbuf[i4].at[r]], sem_sc[g]).wait()

    def scale(g, i4):
        def body(m, _):
            vv = vbuf[i4][pl.ds(m * L, L)]
            for q in range(L):
                gbuf[g][m * L + q] = gbuf[g][m * L + q] * vv[q]
            return 0
        lax.fori_loop(0, CHUNK // L, body, 0)

    for p in range(NPASS):
        j = p * NC + c

        def zfill(k, _):
            gb0[k] = jnp.zeros((L,), jnp.float32)
            return 0
        lax.fori_loop(0, CHUNK, zfill, 0)
        for z in range(ROWS_PER_TILE // CHUNK):
            pltpu.sync_copy(gb0, acc.at[pl.ds(s * ROWS_PER_TILE + z * CHUNK, CHUNK)])
        plsc.subcore_barrier()

        for k0 in range(3):
            loads_fire(k0, k0)
        loads_wait(0)
        gathers_fire(j, 0, 0)

        @pl.loop(0, NITER, step=GSETS * ISETS // 2)
        def _(i):
            for d in range(GSETS * ISETS // 2):
                k = i + d
                g = d % GSETS
                gn = (d + 1) % GSETS
                i4 = d % ISETS
                i4n = (d + 1) % ISETS
                i4p = (d + 3) % ISETS

                @pl.when(k >= 1)
                def _():
                    scatters_wait(gn, i4p)

                @pl.when(k + 3 < NITER)
                def _():
                    loads_fire(k + 3, i4p)

                @pl.when(k + 1 < NITER)
                def _():
                    loads_wait(i4n)
                    gathers_fire(j, gn, i4n)

                gathers_wait(j, g, i4)
                scale(g, i4)
                scatters_fire(g, i4)

        scatters_wait((NITER - 1) % GSETS, (NITER - 1) % ISETS)
        plsc.subcore_barrier()

        pltpu.sync_copy(acc.at[pl.ds(s * ROWS_PER_TILE, ROWS_PER_TILE)],
                        out.at[j].at[pl.ds(s * ROWS_PER_TILE, ROWS_PER_TILE)])


@jax.jit
def _unpool(xc, cols2, rows2, vals):
    mesh = plsc.VectorSubcoreMesh(core_axis_name="c", subcore_axis_name="s",
                                  num_cores=NC, num_subcores=NS)
    k = pl.kernel(
        _body,
        out_type=jax.ShapeDtypeStruct((NCHUNKS, N, L), jnp.float32),
        mesh=mesh,
        compiler_params=pltpu.CompilerParams(use_tc_tiling_on_sc=False),
        scratch_types=(
            [pltpu.VMEM((NROWS, IDXROW), jnp.int32)] * ISETS
            + [pltpu.VMEM((NROWS, IDXROW), jnp.int32)] * ISETS
            + [pltpu.VMEM((CHUNK,), jnp.float32)] * ISETS
            + [pltpu.VMEM((CHUNK, L), jnp.float32)] * GSETS
            + [pltpu.VMEM_SHARED((N, L), jnp.float32)]
            + [pltpu.SemaphoreType.DMA] * (ISETS + 2 * GSETS)
        ),
    )
    return k(xc, cols2, rows2, vals)


def kernel(x, unpool_rows, unpool_cols, unpool_vals):
    xc = x.reshape(N, NCHUNKS, L).transpose(1, 0, 2)
    cols2 = unpool_cols.reshape(NNZ // IDXROW, IDXROW)
    rows2 = unpool_rows.reshape(NNZ // IDXROW, IDXROW)
    out4 = _unpool(xc, cols2, rows2, unpool_vals)
    return out4.transpose(1, 0, 2).reshape(N, D)
